# baseline (speedup 1.0000x reference)
import functools
import jax
import jax.numpy as jnp
from jax import lax
from jax.experimental import pallas as pl
from jax.experimental.pallas import tpu as pltpu
from jax.experimental.pallas import tpu_sc as plsc

_NR, _NC = 2048, 8192
_NW = 32
_RPW = _NR // _NW
_CR = 8
_NCH = _RPW // _CR
_L = 16

_DNUMS = lax.GatherDimensionNumbers(
    offset_dims=(), collapsed_slice_dims=(0,), start_index_map=(0,))


def _sc_body(w_hbm, t_hbm, out_hbm, s_v, sbc, buf, sem_in, sem_out):
    wid = lax.axis_index("s") * 2 + lax.axis_index("c")
    base = wid * _RPW

    pltpu.sync_copy(t_hbm.at[pl.ds(base, _RPW)], s_v)
    for g in range(_RPW // _L):
        t16 = s_v[pl.ds(g * _L, _L)]
        sg = 1.0 / (1.0 + jnp.exp(-t16))
        s_v[pl.ds(g * _L, _L)] = sg
        for lane in range(_L):
            sbc[pl.ds((g * _L + lane) * _L, _L)] = lax.gather(
                sg, jnp.full((_L, 1), lane, jnp.int32), _DNUMS, (1,),
                mode=lax.GatherScatterMode.PROMISE_IN_BOUNDS)

    def chunk(c, carry):
        rowbase = base + c * _CR
        pltpu.async_copy(w_hbm.at[pl.ds(rowbase, _CR), :], buf, sem_in).wait()
        for r in range(_CR):
            s16 = sbc[pl.ds((c * _CR + r) * _L, _L)]
            ns16 = -s16

            @plsc.parallel_loop(0, _NC // _L, step=1, unroll=8)
            def inner(i):
                v = buf[r, pl.ds(i * _L, _L)]
                buf[r, pl.ds(i * _L, _L)] = v - jnp.minimum(
                    jnp.maximum(v, ns16), s16)
        pltpu.async_copy(buf, out_hbm.at[pl.ds(rowbase, _CR), :], sem_out).wait()
        return carry

    lax.fori_loop(0, _NCH, chunk, 0)


def sc_kernel(weight, threshold):
    mesh = plsc.VectorSubcoreMesh(core_axis_name="c", subcore_axis_name="s")
    k = functools.partial(
        pl.kernel,
        mesh=mesh,
        out_type=jax.ShapeDtypeStruct((_NR, _NC), jnp.float32),
        scratch_types=[
            pltpu.VMEM((_RPW,), jnp.float32),
            pltpu.VMEM((_RPW * _L,), jnp.float32),
            pltpu.VMEM((_CR, _NC), jnp.float32),
            pltpu.SemaphoreType.DMA,
            pltpu.SemaphoreType.DMA,
        ],
    )(_sc_body)
    return k(weight, threshold.reshape(-1))


# speedup vs baseline: 2.0991x; 2.0991x over previous
import jax
import jax.numpy as jnp
from jax.experimental import pallas as pl
from jax.experimental.pallas import tpu as pltpu

_NR, _NC = 2048, 8192
_CR = 64
_NBUF = 8

_CHUNKS = []
_row = 0
for _r in [16, 16, 16, 16]:
    _CHUNKS.append((_row, _r))
    _row += _r
while _row < _NR - 64:
    _CHUNKS.append((_row, _CR))
    _row += _CR
for _r in [16, 16, 16, 16]:
    _CHUNKS.append((_row, _r))
    _row += _r
assert _row == _NR
_NCH = len(_CHUNKS)


def _body(w_hbm, t_ref, o_hbm, ibufs, obufs, isems, osems, s_ref):
    def in_copy(c):
        row, nr = _CHUNKS[c]
        k = c % _NBUF
        return pltpu.make_async_copy(
            w_hbm.at[pl.ds(row, nr), :], ibufs.at[k, pl.ds(0, nr)],
            isems.at[k])

    def out_copy(c):
        row, nr = _CHUNKS[c]
        k = c % _NBUF
        return pltpu.make_async_copy(
            obufs.at[k, pl.ds(0, nr)], o_hbm.at[pl.ds(row, nr), :],
            osems.at[k])

    for c in range(_NBUF):
        in_copy(c).start()

    s_ref[:] = jax.nn.sigmoid(t_ref[:]).reshape(_NR, 1)

    for c in range(_NCH):
        row, nr = _CHUNKS[c]
        k = c % _NBUF
        in_copy(c).wait()
        if c >= _NBUF:
            out_copy(c - _NBUF).wait()
        w = ibufs[k, pl.ds(0, nr)]
        s = s_ref[pl.ds(row, nr), :]
        obufs[k, pl.ds(0, nr)] = w - jnp.minimum(jnp.maximum(w, -s), s)
        out_copy(c).start()
        if c + _NBUF < _NCH:
            in_copy(c + _NBUF).start()

    for c in range(_NCH - _NBUF, _NCH):
        out_copy(c).wait()


def kernel(weight, threshold):
    return pl.pallas_call(
        _body,
        in_specs=[
            pl.BlockSpec(memory_space=pltpu.HBM),
            pl.BlockSpec(memory_space=pltpu.VMEM),
        ],
        out_specs=pl.BlockSpec(memory_space=pltpu.HBM),
        out_shape=jax.ShapeDtypeStruct((_NR, _NC), weight.dtype),
        scratch_shapes=[
            pltpu.VMEM((_NBUF, _CR, _NC), jnp.float32),
            pltpu.VMEM((_NBUF, _CR, _NC), jnp.float32),
            pltpu.SemaphoreType.DMA((_NBUF,)),
            pltpu.SemaphoreType.DMA((_NBUF,)),
            pltpu.VMEM((_NR, 1), jnp.float32),
        ],
    )(weight, threshold.reshape(1, _NR))


# speedup vs baseline: 2.1033x; 1.0020x over previous
import jax
import jax.numpy as jnp
from jax.experimental import pallas as pl
from jax.experimental.pallas import tpu as pltpu

_NR, _NC = 2048, 8192
_CR = 64
_NBUF = 8

_CHUNKS = []
_row = 0
for _r in [16, 16, 16, 16]:
    _CHUNKS.append((_row, _r))
    _row += _r
while _row < _NR - 64:
    _CHUNKS.append((_row, _CR))
    _row += _CR
for _r in [16, 16, 8, 8, 8, 8]:
    _CHUNKS.append((_row, _r))
    _row += _r
assert _row == _NR
_NCH = len(_CHUNKS)


def _body(w_hbm, t_ref, o_hbm, ibufs, obufs, isems, osems, s_ref):
    def in_copy(c):
        row, nr = _CHUNKS[c]
        k = c % _NBUF
        return pltpu.make_async_copy(
            w_hbm.at[pl.ds(row, nr), :], ibufs.at[k, pl.ds(0, nr)],
            isems.at[k])

    def out_copy(c):
        row, nr = _CHUNKS[c]
        k = c % _NBUF
        return pltpu.make_async_copy(
            obufs.at[k, pl.ds(0, nr)], o_hbm.at[pl.ds(row, nr), :],
            osems.at[k])

    for c in range(_NBUF):
        in_copy(c).start()

    s_ref[:] = jax.nn.sigmoid(t_ref[:]).reshape(_NR, 1)

    for c in range(_NCH):
        row, nr = _CHUNKS[c]
        k = c % _NBUF
        in_copy(c).wait()
        if c >= _NBUF:
            out_copy(c - _NBUF).wait()
        w = ibufs[k, pl.ds(0, nr)]
        s = s_ref[pl.ds(row, nr), :]
        obufs[k, pl.ds(0, nr)] = w - jnp.minimum(jnp.maximum(w, -s), s)
        out_copy(c).start()
        if c + _NBUF < _NCH:
            in_copy(c + _NBUF).start()

    for c in range(_NCH - _NBUF, _NCH):
        out_copy(c).wait()


def kernel(weight, threshold):
    return pl.pallas_call(
        _body,
        in_specs=[
            pl.BlockSpec(memory_space=pltpu.HBM),
            pl.BlockSpec(memory_space=pltpu.VMEM),
        ],
        out_specs=pl.BlockSpec(memory_space=pltpu.HBM),
        out_shape=jax.ShapeDtypeStruct((_NR, _NC), weight.dtype),
        scratch_shapes=[
            pltpu.VMEM((_NBUF, _CR, _NC), jnp.float32),
            pltpu.VMEM((_NBUF, _CR, _NC), jnp.float32),
            pltpu.SemaphoreType.DMA((_NBUF,)),
            pltpu.SemaphoreType.DMA((_NBUF,)),
            pltpu.VMEM((_NR, 1), jnp.float32),
        ],
    )(weight, threshold.reshape(1, _NR))
